# Initial kernel scaffold; baseline (speedup 1.0000x reference)
import jax
import jax.numpy as jnp
from jax.experimental import pallas as pl


def kernel(features, x, Adj_param, W1, b1, W2, b2):
    raise NotImplementedError("write your pallas kernel here")


# speedup vs baseline: 1.5046x; 1.5046x over previous
import jax
import jax.numpy as jnp
from jax.experimental import pallas as pl
from jax.experimental.pallas import tpu as pltpu

_N = 4096
_IN = 512
_HID = 256
_OUT = 512
_EOS = 1e-10

_B1 = 512
_B2 = 512
_B4 = 512


def _deg_xw1_kernel(a_ref, x_ref, w1_ref, b1_ref, deg_ref, xw1_ref):
    i = pl.program_id(0)
    a = jnp.maximum(a_ref[...], 0.0)
    colpart = jnp.sum(a, axis=0, keepdims=True)
    rowpart = jnp.sum(a, axis=1, keepdims=True)

    @pl.when(i == 0)
    def _init():
        deg_ref[...] = jnp.zeros_like(deg_ref)

    deg_ref[...] += colpart
    deg_ref[:, pl.ds(i * _B1, _B1)] += rowpart.reshape(1, _B1)
    xw1_ref[...] = (
        jnp.dot(x_ref[...], w1_ref[...], preferred_element_type=jnp.float32)
        + b1_ref[...]
    )


def _norm_mm1_kernel(degi_ref, degk_ref, a_ref, at_ref, xw1_ref, adjn_ref, h_ref):
    k = pl.program_id(1)
    nk = pl.num_programs(1)
    di = 1.0 / (jnp.sqrt(degi_ref[...] * 0.5) + _EOS)
    dk = 1.0 / (jnp.sqrt(degk_ref[...] * 0.5) + _EOS)
    s = 0.5 * (jnp.maximum(a_ref[...], 0.0)
               + jnp.maximum(at_ref[...], 0.0).T)
    adjn = di.reshape(_B2, 1) * s * dk
    adjn_ref[...] = adjn
    part = jnp.dot(adjn, xw1_ref[...], preferred_element_type=jnp.float32)

    @pl.when(k == 0)
    def _first():
        h_ref[...] = part

    @pl.when(k > 0)
    def _rest():
        h_ref[...] += part

    @pl.when(k == nk - 1)
    def _last():
        h_ref[...] = jnp.maximum(h_ref[...], 0.0)


def _hw2_kernel(h_ref, w2_ref, b2_ref, hw2_ref):
    hw2_ref[...] = (
        jnp.dot(h_ref[...], w2_ref[...], preferred_element_type=jnp.float32)
        + b2_ref[...]
    )


def _mm2_kernel(adjn_ref, hw2_ref, out_ref):
    k = pl.program_id(1)
    part = jnp.dot(adjn_ref[...], hw2_ref[...], preferred_element_type=jnp.float32)

    @pl.when(k == 0)
    def _first():
        out_ref[...] = part

    @pl.when(k > 0)
    def _rest():
        out_ref[...] += part


def kernel(features, x, Adj_param, W1, b1, W2, b2):
    del features
    b1r = b1.reshape(1, _HID)
    b2r = b2.reshape(1, _OUT)

    n1 = _N // _B1
    degsum, xw1 = pl.pallas_call(
        _deg_xw1_kernel,
        grid=(n1,),
        in_specs=[
            pl.BlockSpec((_B1, _N), lambda i: (i, 0)),
            pl.BlockSpec((_B1, _IN), lambda i: (i, 0)),
            pl.BlockSpec((_IN, _HID), lambda i: (0, 0)),
            pl.BlockSpec((1, _HID), lambda i: (0, 0)),
        ],
        out_specs=[
            pl.BlockSpec((1, _N), lambda i: (0, 0)),
            pl.BlockSpec((_B1, _HID), lambda i: (i, 0)),
        ],
        out_shape=[
            jax.ShapeDtypeStruct((1, _N), jnp.float32),
            jax.ShapeDtypeStruct((_N, _HID), jnp.float32),
        ],
    )(Adj_param, x, W1, b1r)

    n2 = _N // _B2
    adjn, h = pl.pallas_call(
        _norm_mm1_kernel,
        grid=(n2, n2),
        in_specs=[
            pl.BlockSpec((1, _B2), lambda i, k: (0, i)),
            pl.BlockSpec((1, _B2), lambda i, k: (0, k)),
            pl.BlockSpec((_B2, _B2), lambda i, k: (i, k)),
            pl.BlockSpec((_B2, _B2), lambda i, k: (k, i)),
            pl.BlockSpec((_B2, _HID), lambda i, k: (k, 0)),
        ],
        out_specs=[
            pl.BlockSpec((_B2, _B2), lambda i, k: (i, k)),
            pl.BlockSpec((_B2, _HID), lambda i, k: (i, 0)),
        ],
        out_shape=[
            jax.ShapeDtypeStruct((_N, _N), jnp.float32),
            jax.ShapeDtypeStruct((_N, _HID), jnp.float32),
        ],
    )(degsum, degsum, Adj_param, Adj_param, xw1)

    hw2 = pl.pallas_call(
        _hw2_kernel,
        in_specs=[
            pl.BlockSpec((_N, _HID), lambda: (0, 0)),
            pl.BlockSpec((_HID, _OUT), lambda: (0, 0)),
            pl.BlockSpec((1, _OUT), lambda: (0, 0)),
        ],
        out_specs=pl.BlockSpec((_N, _OUT), lambda: (0, 0)),
        out_shape=jax.ShapeDtypeStruct((_N, _OUT), jnp.float32),
    )(h, W2, b2r)

    n4 = _N // _B4
    out = pl.pallas_call(
        _mm2_kernel,
        grid=(n4, n4),
        in_specs=[
            pl.BlockSpec((_B4, _B4), lambda i, k: (i, k)),
            pl.BlockSpec((_B4, _OUT), lambda i, k: (k, 0)),
        ],
        out_specs=pl.BlockSpec((_B4, _OUT), lambda i, k: (i, 0)),
        out_shape=jax.ShapeDtypeStruct((_N, _OUT), jnp.float32),
    )(adjn, hw2)

    return (out, adjn)


# speedup vs baseline: 1.5767x; 1.0479x over previous
import jax
import jax.numpy as jnp
from jax.experimental import pallas as pl
from jax.experimental.pallas import tpu as pltpu

_N = 4096
_IN = 512
_HID = 256
_OUT = 512
_EOS = 1e-10

_B1 = 512
_B2 = 512
_B4 = 512


def _deg_xw1_kernel(a_ref, x_ref, w1_ref, b1_ref, deg_ref, xw1_ref):
    i = pl.program_id(0)
    a = jnp.maximum(a_ref[...], 0.0)
    colpart = jnp.sum(a, axis=0, keepdims=True)
    rowpart = jnp.sum(a, axis=1, keepdims=True)

    @pl.when(i == 0)
    def _init():
        deg_ref[...] = jnp.zeros_like(deg_ref)

    deg_ref[...] += colpart
    deg_ref[:, pl.ds(i * _B1, _B1)] += rowpart.reshape(1, _B1)
    xw1 = (
        jnp.dot(x_ref[...], w1_ref[...], preferred_element_type=jnp.float32)
        + b1_ref[...]
    )
    xw1_ref[...] = xw1.astype(jnp.bfloat16)


def _norm_mm1_kernel(degi_ref, degk_ref, a_ref, at_ref, xw1_ref,
                     adjn_ref, h_ref, acc_ref):
    k = pl.program_id(1)
    nk = pl.num_programs(1)
    di = 1.0 / (jnp.sqrt(degi_ref[...] * 0.5) + _EOS)
    dk = 1.0 / (jnp.sqrt(degk_ref[...] * 0.5) + _EOS)
    s = 0.5 * (jnp.maximum(a_ref[...], 0.0)
               + jnp.maximum(at_ref[...], 0.0).T)
    adjn = di.reshape(_B2, 1) * s * dk
    adjn_ref[...] = adjn
    part = jnp.dot(adjn.astype(jnp.bfloat16), xw1_ref[...],
                   preferred_element_type=jnp.float32)

    @pl.when(k == 0)
    def _first():
        acc_ref[...] = part

    @pl.when(k > 0)
    def _rest():
        acc_ref[...] += part

    @pl.when(k == nk - 1)
    def _last():
        h_ref[...] = jnp.maximum(acc_ref[...], 0.0).astype(jnp.bfloat16)


def _hw2_kernel(h_ref, w2_ref, b2_ref, hw2_ref):
    hw2 = (
        jnp.dot(h_ref[...], w2_ref[...], preferred_element_type=jnp.float32)
        + b2_ref[...]
    )
    hw2_ref[...] = hw2.astype(jnp.bfloat16)


def _mm2_kernel(adjn_ref, hw2_ref, out_ref):
    k = pl.program_id(1)
    part = jnp.dot(adjn_ref[...].astype(jnp.bfloat16), hw2_ref[...],
                   preferred_element_type=jnp.float32)

    @pl.when(k == 0)
    def _first():
        out_ref[...] = part

    @pl.when(k > 0)
    def _rest():
        out_ref[...] += part


def kernel(features, x, Adj_param, W1, b1, W2, b2):
    del features
    b1r = b1.reshape(1, _HID)
    b2r = b2.reshape(1, _OUT)

    n1 = _N // _B1
    degsum, xw1 = pl.pallas_call(
        _deg_xw1_kernel,
        grid=(n1,),
        in_specs=[
            pl.BlockSpec((_B1, _N), lambda i: (i, 0)),
            pl.BlockSpec((_B1, _IN), lambda i: (i, 0)),
            pl.BlockSpec((_IN, _HID), lambda i: (0, 0)),
            pl.BlockSpec((1, _HID), lambda i: (0, 0)),
        ],
        out_specs=[
            pl.BlockSpec((1, _N), lambda i: (0, 0)),
            pl.BlockSpec((_B1, _HID), lambda i: (i, 0)),
        ],
        out_shape=[
            jax.ShapeDtypeStruct((1, _N), jnp.float32),
            jax.ShapeDtypeStruct((_N, _HID), jnp.bfloat16),
        ],
    )(Adj_param, x, W1, b1r)

    n2 = _N // _B2
    adjn, h = pl.pallas_call(
        _norm_mm1_kernel,
        grid=(n2, n2),
        in_specs=[
            pl.BlockSpec((1, _B2), lambda i, k: (0, i)),
            pl.BlockSpec((1, _B2), lambda i, k: (0, k)),
            pl.BlockSpec((_B2, _B2), lambda i, k: (i, k)),
            pl.BlockSpec((_B2, _B2), lambda i, k: (k, i)),
            pl.BlockSpec((_B2, _HID), lambda i, k: (k, 0)),
        ],
        out_specs=[
            pl.BlockSpec((_B2, _B2), lambda i, k: (i, k)),
            pl.BlockSpec((_B2, _HID), lambda i, k: (i, 0)),
        ],
        out_shape=[
            jax.ShapeDtypeStruct((_N, _N), jnp.float32),
            jax.ShapeDtypeStruct((_N, _HID), jnp.bfloat16),
        ],
        scratch_shapes=[pltpu.VMEM((_B2, _HID), jnp.float32)],
    )(degsum, degsum, Adj_param, Adj_param, xw1)

    hw2 = pl.pallas_call(
        _hw2_kernel,
        in_specs=[
            pl.BlockSpec((_N, _HID), lambda: (0, 0)),
            pl.BlockSpec((_HID, _OUT), lambda: (0, 0)),
            pl.BlockSpec((1, _OUT), lambda: (0, 0)),
        ],
        out_specs=pl.BlockSpec((_N, _OUT), lambda: (0, 0)),
        out_shape=jax.ShapeDtypeStruct((_N, _OUT), jnp.bfloat16),
    )(h, W2.astype(jnp.bfloat16), b2r)

    n4 = _N // _B4
    out = pl.pallas_call(
        _mm2_kernel,
        grid=(n4, n4),
        in_specs=[
            pl.BlockSpec((_B4, _B4), lambda i, k: (i, k)),
            pl.BlockSpec((_B4, _OUT), lambda i, k: (k, 0)),
        ],
        out_specs=pl.BlockSpec((_B4, _OUT), lambda i, k: (i, 0)),
        out_shape=jax.ShapeDtypeStruct((_N, _OUT), jnp.float32),
    )(adjn, hw2)

    return (out, adjn)
